# initial kernel scaffold (unmeasured)
import jax
import jax.numpy as jnp
from jax import lax
from jax.experimental import pallas as pl
from jax.experimental.pallas import tpu as pltpu

N_DEV = 16
E4M3_MAX = 448.0


def _quant_dequant_e4m3(z, scale):
    t = z / scale
    t = jnp.clip(t, -E4M3_MAX, E4M3_MAX)
    u = lax.bitcast_convert_type(t, jnp.uint32)
    u = (u + jnp.uint32(0x7FFFF) + ((u >> jnp.uint32(20)) & jnp.uint32(1))) & jnp.uint32(
        0xFFF00000
    )
    t = lax.bitcast_convert_type(u, jnp.float32)
    return t * scale


def kernel(x, w_mat):
    m_per, k = x.shape
    _, n = w_mat.shape
    n_per = n // N_DEV

    def body(
        x_ref,
        w_ref,
        out_ref,
        y_src,
        data_recv,
        amax_src,
        amax_recv,
        dsend,
        drecv,
        asend,
        arecv,
    ):
        my = lax.axis_index("i")

        xb = x_ref[...].astype(jnp.bfloat16)
        wb = w_ref[...].astype(jnp.bfloat16)
        y = jnp.dot(xb, wb, preferred_element_type=jnp.float32)
        amax = jnp.max(jnp.abs(y))

        y_src[...] = y.astype(jnp.bfloat16)
        amax_src[...] = jnp.full((8, 128), amax, jnp.float32)
        amax_recv[0, :, :] = jnp.full((8, 128), amax, jnp.float32)

        sends = []
        for d in range(1, N_DEV):
            j = lax.rem(my + d, N_DEV)
            c = pltpu.make_async_remote_copy(
                src_ref=y_src.at[:, pl.ds(j * n_per, n_per)],
                dst_ref=data_recv.at[d],
                send_sem=dsend.at[d],
                recv_sem=drecv.at[d],
                device_id=(j,),
                device_id_type=pl.DeviceIdType.MESH,
            )
            c.start()
            a = pltpu.make_async_remote_copy(
                src_ref=amax_src,
                dst_ref=amax_recv.at[d],
                send_sem=asend.at[d],
                recv_sem=arecv.at[d],
                device_id=(j,),
                device_id_type=pl.DeviceIdType.MESH,
            )
            a.start()
            sends.append((c, a))

        for _, a in sends:
            a.wait_recv()
        gmax = jnp.max(amax_recv[...])
        scale = gmax / E4M3_MAX

        own = lax.dynamic_slice(y, (0, my * n_per), (m_per, n_per))
        out_ref[pl.ds(my * m_per, m_per), :] = _quant_dequant_e4m3(own, scale)

        for d, (c, _) in zip(range(1, N_DEV), sends):
            c.wait_recv()
            origin = lax.rem(my - d + N_DEV, N_DEV)
            z = data_recv[d].astype(jnp.float32)
            out_ref[pl.ds(origin * m_per, m_per), :] = _quant_dequant_e4m3(z, scale)

        for c, a in sends:
            c.wait_send()
            a.wait_send()

    return pl.pallas_call(
        body,
        out_shape=jax.ShapeDtypeStruct((N_DEV * m_per, n_per), jnp.float32),
        in_specs=[
            pl.BlockSpec(memory_space=pltpu.VMEM),
            pl.BlockSpec(memory_space=pltpu.VMEM),
        ],
        out_specs=pl.BlockSpec(memory_space=pltpu.VMEM),
        scratch_shapes=[
            pltpu.VMEM((m_per, n), jnp.bfloat16),
            pltpu.VMEM((N_DEV, m_per, n_per), jnp.bfloat16),
            pltpu.VMEM((8, 128), jnp.float32),
            pltpu.VMEM((N_DEV, 8, 128), jnp.float32),
            pltpu.SemaphoreType.DMA((N_DEV,)),
            pltpu.SemaphoreType.DMA((N_DEV,)),
            pltpu.SemaphoreType.DMA((N_DEV,)),
            pltpu.SemaphoreType.DMA((N_DEV,)),
        ],
    )(x, w_mat)


# baseline (device time: 39271 ns/iter reference)
import jax
import jax.numpy as jnp
from jax import lax
from jax.experimental import pallas as pl
from jax.experimental.pallas import tpu as pltpu

N_DEV = 16
E4M3_MAX = 448.0


def _quant_dequant_e4m3(z, scale):
    t = z / scale
    t = jnp.clip(t, -E4M3_MAX, E4M3_MAX)
    u = lax.bitcast_convert_type(t, jnp.uint32)
    u = (u + jnp.uint32(0x7FFFF) + ((u >> jnp.uint32(20)) & jnp.uint32(1))) & jnp.uint32(
        0xFFF00000
    )
    t = lax.bitcast_convert_type(u, jnp.float32)
    return t * scale


def kernel(x, w_mat):
    m_per, k = x.shape
    _, n = w_mat.shape
    n_per = n // N_DEV

    def body(
        x_ref,
        w_ref,
        out_ref,
        wbuf,
        y_f32,
        y_src,
        data_recv,
        amax_src,
        amax_recv,
        wsem,
        dsend,
        drecv,
        asend,
        arecv,
    ):
        my = lax.axis_index("i")

        def w_block_copy(t):
            jj = lax.rem(my + t + 1, N_DEV) if t < N_DEV - 1 else my
            return jj, pltpu.make_async_copy(
                w_ref.at[:, pl.ds(jj * n_per, n_per)],
                wbuf.at[t % 2],
                wsem.at[t % 2],
            )

        _, cp0 = w_block_copy(0)
        cp0.start()

        xb = x_ref[...].astype(jnp.bfloat16)

        data_rdmas = []
        amax_partials = []
        for t in range(N_DEV):
            jj, cp = w_block_copy(t)
            if t + 1 < N_DEV:
                jj_next, cp_next = w_block_copy(t + 1)
                cp_next.start()
            cp.wait()
            wb = wbuf[t % 2].astype(jnp.bfloat16)
            yb = jnp.dot(xb, wb, preferred_element_type=jnp.float32)
            amax_partials.append(jnp.max(jnp.abs(yb)))
            y_f32[:, pl.ds(jj * n_per, n_per)] = yb
            y_src[:, pl.ds(jj * n_per, n_per)] = yb.astype(jnp.bfloat16)
            if t < N_DEV - 1:
                d = t + 1
                c = pltpu.make_async_remote_copy(
                    src_ref=y_src.at[:, pl.ds(jj * n_per, n_per)],
                    dst_ref=data_recv.at[d],
                    send_sem=dsend.at[d],
                    recv_sem=drecv.at[d],
                    device_id=(jj,),
                    device_id_type=pl.DeviceIdType.MESH,
                )
                c.start()
                data_rdmas.append(c)

        amax = amax_partials[0]
        for p in amax_partials[1:]:
            amax = jnp.maximum(amax, p)
        amax_src[...] = jnp.full((8, 128), amax, jnp.float32)
        amax_recv[0, :, :] = jnp.full((8, 128), amax, jnp.float32)

        amax_rdmas = []
        for d in range(1, N_DEV):
            j = lax.rem(my + d, N_DEV)
            a = pltpu.make_async_remote_copy(
                src_ref=amax_src,
                dst_ref=amax_recv.at[d],
                send_sem=asend.at[d],
                recv_sem=arecv.at[d],
                device_id=(j,),
                device_id_type=pl.DeviceIdType.MESH,
            )
            a.start()
            amax_rdmas.append(a)

        for a in amax_rdmas:
            a.wait_recv()
        gmax = jnp.max(amax_recv[...])
        scale = gmax / E4M3_MAX

        own = y_f32[:, pl.ds(my * n_per, n_per)]
        out_ref[pl.ds(my * m_per, m_per), :] = _quant_dequant_e4m3(own, scale)

        for d, c in zip(range(1, N_DEV), data_rdmas):
            c.wait_recv()
            origin = lax.rem(my - d + N_DEV, N_DEV)
            z = data_recv[d].astype(jnp.float32)
            out_ref[pl.ds(origin * m_per, m_per), :] = _quant_dequant_e4m3(z, scale)

        for c in data_rdmas:
            c.wait_send()
        for a in amax_rdmas:
            a.wait_send()

    return pl.pallas_call(
        body,
        out_shape=jax.ShapeDtypeStruct((N_DEV * m_per, n_per), jnp.float32),
        in_specs=[
            pl.BlockSpec(memory_space=pltpu.VMEM),
            pl.BlockSpec(memory_space=pltpu.MemorySpace.HBM),
        ],
        out_specs=pl.BlockSpec(memory_space=pltpu.VMEM),
        scratch_shapes=[
            pltpu.VMEM((2, k, n_per), jnp.float32),
            pltpu.VMEM((m_per, n), jnp.float32),
            pltpu.VMEM((m_per, n), jnp.bfloat16),
            pltpu.VMEM((N_DEV, m_per, n_per), jnp.bfloat16),
            pltpu.VMEM((8, 128), jnp.float32),
            pltpu.VMEM((N_DEV, 8, 128), jnp.float32),
            pltpu.SemaphoreType.DMA((2,)),
            pltpu.SemaphoreType.DMA((N_DEV,)),
            pltpu.SemaphoreType.DMA((N_DEV,)),
            pltpu.SemaphoreType.DMA((N_DEV,)),
            pltpu.SemaphoreType.DMA((N_DEV,)),
        ],
    )(x, w_mat)


# device time: 35992 ns/iter; 1.0911x vs baseline; 1.0911x over previous
import jax
import jax.numpy as jnp
from jax import lax
from jax.experimental import pallas as pl
from jax.experimental.pallas import tpu as pltpu

N_DEV = 16
NBUF = 8
E4M3_MAX = 448.0


def _quant_dequant_e4m3(z, inv_scale, scale):
    t = z * inv_scale
    t = jnp.clip(t, -E4M3_MAX, E4M3_MAX)
    u = lax.bitcast_convert_type(t, jnp.uint32)
    u = (u + jnp.uint32(0x7FFFF) + ((u >> jnp.uint32(20)) & jnp.uint32(1))) & jnp.uint32(
        0xFFF00000
    )
    t = lax.bitcast_convert_type(u, jnp.float32)
    return t * scale


def kernel(x, w_mat):
    m_per, k = x.shape
    _, n = w_mat.shape
    n_per = n // N_DEV

    def body(
        x_ref,
        w_ref,
        out_ref,
        wbuf,
        y_f32,
        y_src,
        data_recv,
        amax_src,
        amax_recv,
        wsem,
        dsend,
        drecv,
        asend,
        arecv,
    ):
        my = lax.axis_index("i")

        def w_block_copy(t):
            jj = lax.rem(my + t + 1, N_DEV) if t < N_DEV - 1 else my
            return jj, pltpu.make_async_copy(
                w_ref.at[:, pl.ds(jj * n_per, n_per)],
                wbuf.at[t % NBUF],
                wsem.at[t % NBUF],
            )

        for t in range(NBUF - 1):
            _, cp = w_block_copy(t)
            cp.start()

        xb = x_ref[...].astype(jnp.bfloat16)

        data_rdmas = []
        amax_partials = []
        for t in range(N_DEV):
            jj, cp = w_block_copy(t)
            if t + NBUF - 1 < N_DEV:
                _, cp_next = w_block_copy(t + NBUF - 1)
                cp_next.start()
            cp.wait()
            wb = wbuf[t % NBUF].astype(jnp.bfloat16)
            yb = jnp.dot(xb, wb, preferred_element_type=jnp.float32)
            amax_partials.append(jnp.max(jnp.abs(yb)))
            y_f32[:, pl.ds(jj * n_per, n_per)] = yb
            y_src[:, pl.ds(jj * n_per, n_per)] = yb.astype(jnp.bfloat16)
            if t < N_DEV - 1:
                d = t + 1
                c = pltpu.make_async_remote_copy(
                    src_ref=y_src.at[:, pl.ds(jj * n_per, n_per)],
                    dst_ref=data_recv.at[d],
                    send_sem=dsend.at[d],
                    recv_sem=drecv.at[d],
                    device_id=(jj,),
                    device_id_type=pl.DeviceIdType.MESH,
                )
                c.start()
                data_rdmas.append(c)

        amax = amax_partials[0]
        for p in amax_partials[1:]:
            amax = jnp.maximum(amax, p)
        amax_src[...] = jnp.full((8, 128), amax, jnp.float32)
        amax_recv[0, :, :] = jnp.full((8, 128), amax, jnp.float32)

        amax_rdmas = []
        for d in range(1, N_DEV):
            j = lax.rem(my + d, N_DEV)
            a = pltpu.make_async_remote_copy(
                src_ref=amax_src,
                dst_ref=amax_recv.at[d],
                send_sem=asend.at[d],
                recv_sem=arecv.at[d],
                device_id=(j,),
                device_id_type=pl.DeviceIdType.MESH,
            )
            a.start()
            amax_rdmas.append(a)

        for a in amax_rdmas:
            a.wait_recv()
        gmax = jnp.max(amax_recv[...])
        scale = gmax / E4M3_MAX
        inv_scale = E4M3_MAX / gmax

        own = y_f32[:, pl.ds(my * n_per, n_per)]
        out_ref[pl.ds(my * m_per, m_per), :] = _quant_dequant_e4m3(
            own, inv_scale, scale
        )

        for d, c in zip(range(1, N_DEV), data_rdmas):
            c.wait_recv()
            origin = lax.rem(my - d + N_DEV, N_DEV)
            z = data_recv[d].astype(jnp.float32)
            out_ref[pl.ds(origin * m_per, m_per), :] = _quant_dequant_e4m3(
                z, inv_scale, scale
            )

        for c in data_rdmas:
            c.wait_send()
        for a in amax_rdmas:
            a.wait_send()

    return pl.pallas_call(
        body,
        out_shape=jax.ShapeDtypeStruct((N_DEV * m_per, n_per), jnp.float32),
        in_specs=[
            pl.BlockSpec(memory_space=pltpu.VMEM),
            pl.BlockSpec(memory_space=pltpu.MemorySpace.HBM),
        ],
        out_specs=pl.BlockSpec(memory_space=pltpu.VMEM),
        scratch_shapes=[
            pltpu.VMEM((NBUF, k, n_per), jnp.float32),
            pltpu.VMEM((m_per, n), jnp.float32),
            pltpu.VMEM((m_per, n), jnp.bfloat16),
            pltpu.VMEM((N_DEV, m_per, n_per), jnp.bfloat16),
            pltpu.VMEM((8, 128), jnp.float32),
            pltpu.VMEM((N_DEV, 8, 128), jnp.float32),
            pltpu.SemaphoreType.DMA((NBUF,)),
            pltpu.SemaphoreType.DMA((N_DEV,)),
            pltpu.SemaphoreType.DMA((N_DEV,)),
            pltpu.SemaphoreType.DMA((N_DEV,)),
            pltpu.SemaphoreType.DMA((N_DEV,)),
        ],
    )(x, w_mat)


# device time: 34324 ns/iter; 1.1441x vs baseline; 1.0486x over previous
import os

import jax
import jax.numpy as jnp
from jax import lax
from jax.experimental import pallas as pl
from jax.experimental.pallas import tpu as pltpu

N_DEV = 16
NBUF = 8
E4M3_MAX = 448.0
VARIANT = os.environ.get("KVARIANT", "full")


def _quant_dequant_e4m3(z, inv_scale, scale):
    t = z * inv_scale
    t = jnp.clip(t, -E4M3_MAX, E4M3_MAX)
    u = lax.bitcast_convert_type(t, jnp.uint32)
    u = (u + jnp.uint32(0x7FFFF) + ((u >> jnp.uint32(20)) & jnp.uint32(1))) & jnp.uint32(
        0xFFF00000
    )
    t = lax.bitcast_convert_type(u, jnp.float32)
    return t * scale


def kernel(x, w_mat):
    m_per, k = x.shape
    _, n = w_mat.shape
    n_per = n // N_DEV

    def body(
        x_ref,
        w_ref,
        out_ref,
        wbuf,
        y_src,
        recv,
        amax_src,
        amax_recv,
        wsem,
        dsend,
        drecv,
        asend,
        arecv,
    ):
        my = lax.axis_index("i")

        def w_block_copy(t):
            jj = lax.rem(my + t + 1, N_DEV) if t < N_DEV - 1 else my
            return jj, pltpu.make_async_copy(
                w_ref.at[:, pl.ds(jj * n_per, n_per)],
                wbuf.at[t % NBUF],
                wsem.at[t % NBUF],
            )

        if VARIANT != "nogemm":
            for t in range(NBUF - 1):
                _, cp = w_block_copy(t)
                cp.start()

        xb = x_ref[...].astype(jnp.bfloat16)

        data_rdmas = []
        vmax = None
        for t in range(N_DEV):
            jj, cp = w_block_copy(t)
            if VARIANT != "nogemm":
                if t + NBUF - 1 < N_DEV:
                    _, cp_next = w_block_copy(t + NBUF - 1)
                    cp_next.start()
                cp.wait()
                wb = wbuf[t % NBUF].astype(jnp.bfloat16)
                yb = jnp.dot(xb, wb, preferred_element_type=jnp.float32)
            else:
                yb = xb[:, t * n_per : (t + 1) * n_per].astype(jnp.float32)
            pmax = jnp.max(jnp.abs(yb), axis=0, keepdims=True)
            vmax = pmax if vmax is None else jnp.maximum(vmax, pmax)
            if t < N_DEV - 1:
                y_src[:, t * n_per : (t + 1) * n_per] = yb.astype(jnp.bfloat16)
                if VARIANT != "nocomm":
                    d = t + 1
                    c = pltpu.make_async_remote_copy(
                        src_ref=y_src.at[:, pl.ds(t * n_per, n_per)],
                        dst_ref=recv.at[pl.ds(my * m_per, m_per), :],
                        send_sem=dsend.at[d],
                        recv_sem=drecv.at[d],
                        device_id=(jj,),
                        device_id_type=pl.DeviceIdType.MESH,
                    )
                    c.start()
                    data_rdmas.append(c)
            else:
                recv[pl.ds(my * m_per, m_per), :] = yb.astype(jnp.bfloat16)

        amax_src[...] = jnp.zeros((8, 128), jnp.float32) + vmax
        amax_recv[0, :, :] = jnp.zeros((8, 128), jnp.float32) + vmax

        amax_rdmas = []
        for d in range(1, N_DEV) if VARIANT != "nocomm" else []:
            j = lax.rem(my + d, N_DEV)
            a = pltpu.make_async_remote_copy(
                src_ref=amax_src,
                dst_ref=amax_recv.at[d],
                send_sem=asend.at[d],
                recv_sem=arecv.at[d],
                device_id=(j,),
                device_id_type=pl.DeviceIdType.MESH,
            )
            a.start()
            amax_rdmas.append(a)

        for a in amax_rdmas:
            a.wait_recv()
        gmax = jnp.max(amax_recv[...])
        scale = gmax / E4M3_MAX
        inv_scale = E4M3_MAX / gmax

        for c in data_rdmas:
            c.wait_recv()

        out_ref[...] = _quant_dequant_e4m3(
            recv[...].astype(jnp.float32), inv_scale, scale
        )

        for c in data_rdmas:
            c.wait_send()
        for a in amax_rdmas:
            a.wait_send()

    return pl.pallas_call(
        body,
        out_shape=jax.ShapeDtypeStruct((N_DEV * m_per, n_per), jnp.float32),
        in_specs=[
            pl.BlockSpec(memory_space=pltpu.VMEM),
            pl.BlockSpec(memory_space=pltpu.MemorySpace.HBM),
        ],
        out_specs=pl.BlockSpec(memory_space=pltpu.VMEM),
        scratch_shapes=[
            pltpu.VMEM((NBUF, k, n_per), jnp.float32),
            pltpu.VMEM((m_per, n), jnp.bfloat16),
            pltpu.VMEM((N_DEV * m_per, n_per), jnp.bfloat16),
            pltpu.VMEM((8, 128), jnp.float32),
            pltpu.VMEM((N_DEV, 8, 128), jnp.float32),
            pltpu.SemaphoreType.DMA((NBUF,)),
            pltpu.SemaphoreType.DMA((N_DEV,)),
            pltpu.SemaphoreType.DMA((N_DEV,)),
            pltpu.SemaphoreType.DMA((N_DEV,)),
            pltpu.SemaphoreType.DMA((N_DEV,)),
        ],
    )(x, w_mat)
